# baseline (speedup 1.0000x reference)
import jax
import jax.numpy as jnp
import numpy as np
from jax.experimental import pallas as pl

SH_C0 = 0.28209479177387814
SH_C1 = 0.4886025119029199
SH_C2 = [1.0925484305920792, -1.0925484305920792, 0.31539156525252005,
         -1.0925484305920792, 0.5462742152960396]

NUM_CODES = 4096
CODE_DIM = 32
NUM_NEIGHBORS = 16
SH_BASIS_DIM = 9
NUM_POINTS = 8192
QBLK = 256
IDX_MASK = ~4095
INT_MAX = 2147483647


def _tc_body(q_ref, v_ref, cposT_ref, codes_ref, sh_ref, sel_ref,
             out_c_ref, out_s_ref):
    q = q_ref[...]
    cposT = cposT_ref[...]
    acc = None
    for d in range(3):
        diff = q[:, d:d + 1] - cposT[d:d + 1, :]
        acc = diff * diff if acc is None else acc + diff * diff
    mm = jnp.dot(q.astype(jnp.bfloat16), cposT.astype(jnp.bfloat16),
                 preferred_element_type=jnp.float32)
    qq = jnp.sum(q * q, axis=1, keepdims=True)
    cc = jnp.sum(cposT * cposT, axis=0, keepdims=True)
    d2sel = qq + cc - 2.0 * mm
    iota = jax.lax.broadcasted_iota(jnp.int32, (QBLK, NUM_CODES), 1)
    p = jnp.bitwise_or(
        jnp.bitwise_and(jax.lax.bitcast_convert_type(d2sel, jnp.int32),
                        IDX_MASK),
        iota)

    m0 = jnp.min(p, axis=1, keepdims=True)

    def step(_, carry):
        p, m = carry
        p2 = jnp.where(p == m, INT_MAX, p)
        return p2, jnp.min(p2, axis=1, keepdims=True)

    (p, _) = jax.lax.fori_loop(0, NUM_NEIGHBORS, step, (p, m0))
    W = jnp.where(p == INT_MAX, 1.0 / (acc + 1e-16), 0.0)
    wsum = jnp.sum(W, axis=1, keepdims=True)

    qc = jnp.dot(W, codes_ref[...], preferred_element_type=jnp.float32) / wsum
    G = jnp.dot(W, sh_ref[...], preferred_element_type=jnp.float32) / wsum

    v = v_ref[...]
    x = v[:, 0:1]
    y = v[:, 1:2]
    z = v[:, 2:3]
    xx, yy, zz = x * x, y * y, z * z
    shb = [
        jnp.full((QBLK, 1), SH_C0, jnp.float32),
        -SH_C1 * y,
        SH_C1 * z,
        -SH_C1 * x,
        SH_C2[0] * (x * y),
        SH_C2[1] * (y * z),
        SH_C2[2] * (2.0 * zz - xx - yy),
        SH_C2[3] * (x * z),
        SH_C2[4] * (xx - yy),
    ]
    bidx = jax.lax.broadcasted_iota(
        jnp.int32, (QBLK, CODE_DIM * SH_BASIS_DIM), 1) % SH_BASIS_DIM
    M = jnp.zeros((QBLK, CODE_DIM * SH_BASIS_DIM), jnp.float32)
    for b in range(SH_BASIS_DIM):
        M = jnp.where(bidx == b, shb[b], M)
    out_s_ref[...] = jnp.dot(G * M, sel_ref[...],
                             preferred_element_type=jnp.float32)
    out_c_ref[...] = qc


def kernel(codes_position, codes, sh_codes, indices, query_points, viewdirs):
    idx0 = indices[0]
    cposT = codes_position[idx0].T
    codes_sel = codes[idx0]
    sh_sel = sh_codes[idx0]
    q = query_points[0]

    sel_np = np.zeros((CODE_DIM * SH_BASIS_DIM, CODE_DIM), np.float32)
    sel_np[np.arange(CODE_DIM * SH_BASIS_DIM),
           np.arange(CODE_DIM * SH_BASIS_DIM) // SH_BASIS_DIM] = 1.0
    sel = jnp.asarray(sel_np)

    grid = (NUM_POINTS // QBLK,)
    out_c, out_s = pl.pallas_call(
        _tc_body,
        grid=grid,
        in_specs=[
            pl.BlockSpec((QBLK, 3), lambda i: (i, 0)),
            pl.BlockSpec((QBLK, 3), lambda i: (i, 0)),
            pl.BlockSpec((3, NUM_CODES), lambda i: (0, 0)),
            pl.BlockSpec((NUM_CODES, CODE_DIM), lambda i: (0, 0)),
            pl.BlockSpec((NUM_CODES, CODE_DIM * SH_BASIS_DIM), lambda i: (0, 0)),
            pl.BlockSpec((CODE_DIM * SH_BASIS_DIM, CODE_DIM), lambda i: (0, 0)),
        ],
        out_specs=[
            pl.BlockSpec((QBLK, CODE_DIM), lambda i: (i, 0)),
            pl.BlockSpec((QBLK, CODE_DIM), lambda i: (i, 0)),
        ],
        out_shape=[
            jax.ShapeDtypeStruct((NUM_POINTS, CODE_DIM), jnp.float32),
            jax.ShapeDtypeStruct((NUM_POINTS, CODE_DIM), jnp.float32),
        ],
    )(q, viewdirs, cposT, codes_sel, sh_sel, sel)
    return (out_c, out_s)


# speedup vs baseline: 1.2384x; 1.2384x over previous
import jax
import jax.numpy as jnp
import numpy as np
from jax.experimental import pallas as pl

SH_C0 = 0.28209479177387814
SH_C1 = 0.4886025119029199
SH_C2 = [1.0925484305920792, -1.0925484305920792, 0.31539156525252005,
         -1.0925484305920792, 0.5462742152960396]

NUM_CODES = 4096
CODE_DIM = 32
NUM_NEIGHBORS = 16
SH_BASIS_DIM = 9
SH_WIDTH = CODE_DIM * SH_BASIS_DIM
NUM_POINTS = 8192
QBLK = 256
IDX_MASK = ~4095
INT_MAX = 2147483647


def _tc_body(qT_ref, vT_ref, cpos_ref, codesT_ref, shT_ref, sel_ref,
             out_c_ref, out_s_ref):
    qT = qT_ref[...]
    cpos = cpos_ref[...]
    acc = None
    for d in range(3):
        diff = cpos[:, d:d + 1] - qT[d:d + 1, :]
        acc = diff * diff if acc is None else acc + diff * diff
    mm = jnp.dot(cpos.astype(jnp.bfloat16), qT.astype(jnp.bfloat16),
                 preferred_element_type=jnp.float32)
    qq = jnp.sum(qT * qT, axis=0, keepdims=True)
    cc = jnp.sum(cpos * cpos, axis=1, keepdims=True)
    d2sel = qq + cc - 2.0 * mm
    iota = jax.lax.broadcasted_iota(jnp.int32, (NUM_CODES, QBLK), 0)
    p = jnp.bitwise_or(
        jnp.bitwise_and(jax.lax.bitcast_convert_type(d2sel, jnp.int32),
                        IDX_MASK),
        iota)

    m0 = jnp.min(p, axis=0, keepdims=True)

    def step(_, carry):
        p, m = carry
        p2 = jnp.where(p == m, INT_MAX, p)
        return p2, jnp.min(p2, axis=0, keepdims=True)

    (p, _) = jax.lax.fori_loop(0, NUM_NEIGHBORS, step, (p, m0))

    WT = jnp.where(p == INT_MAX, 1.0 / (acc + 1e-16), 0.0)
    wsum = jnp.sum(WT, axis=0, keepdims=True)

    qcT = jnp.dot(codesT_ref[...], WT,
                  preferred_element_type=jnp.float32) / wsum
    GT = jnp.dot(shT_ref[...], WT,
                 preferred_element_type=jnp.float32) / wsum

    vT = vT_ref[...]
    x = vT[0:1, :]
    y = vT[1:2, :]
    z = vT[2:3, :]
    xx, yy, zz = x * x, y * y, z * z
    shb = [
        jnp.full((1, QBLK), SH_C0, jnp.float32),
        -SH_C1 * y,
        SH_C1 * z,
        -SH_C1 * x,
        SH_C2[0] * (x * y),
        SH_C2[1] * (y * z),
        SH_C2[2] * (2.0 * zz - xx - yy),
        SH_C2[3] * (x * z),
        SH_C2[4] * (xx - yy),
    ]
    bidx = jax.lax.broadcasted_iota(jnp.int32, (SH_WIDTH, QBLK), 0) % SH_BASIS_DIM
    MT = jnp.zeros((SH_WIDTH, QBLK), jnp.float32)
    for b in range(SH_BASIS_DIM):
        MT = jnp.where(bidx == b, shb[b], MT)
    out_s_ref[...] = jnp.dot(sel_ref[...], GT * MT,
                             preferred_element_type=jnp.float32)
    out_c_ref[...] = qcT


def kernel(codes_position, codes, sh_codes, indices, query_points, viewdirs):
    idx0 = indices[0]
    cpos = codes_position[idx0]
    codesT = codes[idx0].T
    shT = sh_codes[idx0].T
    qT = query_points[0].T
    vT = viewdirs.T

    sel_np = np.zeros((CODE_DIM, SH_WIDTH), np.float32)
    sel_np[np.arange(SH_WIDTH) // SH_BASIS_DIM, np.arange(SH_WIDTH)] = 1.0
    sel = jnp.asarray(sel_np)

    grid = (NUM_POINTS // QBLK,)
    out_cT, out_sT = pl.pallas_call(
        _tc_body,
        grid=grid,
        in_specs=[
            pl.BlockSpec((3, QBLK), lambda i: (0, i)),
            pl.BlockSpec((3, QBLK), lambda i: (0, i)),
            pl.BlockSpec((NUM_CODES, 3), lambda i: (0, 0)),
            pl.BlockSpec((CODE_DIM, NUM_CODES), lambda i: (0, 0)),
            pl.BlockSpec((SH_WIDTH, NUM_CODES), lambda i: (0, 0)),
            pl.BlockSpec((CODE_DIM, SH_WIDTH), lambda i: (0, 0)),
        ],
        out_specs=[
            pl.BlockSpec((CODE_DIM, QBLK), lambda i: (0, i)),
            pl.BlockSpec((CODE_DIM, QBLK), lambda i: (0, i)),
        ],
        out_shape=[
            jax.ShapeDtypeStruct((CODE_DIM, NUM_POINTS), jnp.float32),
            jax.ShapeDtypeStruct((CODE_DIM, NUM_POINTS), jnp.float32),
        ],
    )(qT, vT, cpos, codesT, shT, sel)
    return (out_cT.T, out_sT.T)


# speedup vs baseline: 2.1073x; 1.7016x over previous
import jax
import jax.numpy as jnp
import numpy as np
from jax.experimental import pallas as pl

SH_C0 = 0.28209479177387814
SH_C1 = 0.4886025119029199
SH_C2 = [1.0925484305920792, -1.0925484305920792, 0.31539156525252005,
         -1.0925484305920792, 0.5462742152960396]

NUM_CODES = 4096
CODE_DIM = 32
NUM_NEIGHBORS = 16
SH_BASIS_DIM = 9
SH_WIDTH = CODE_DIM * SH_BASIS_DIM
NUM_POINTS = 8192
QBLK = 256
IDX_MASK = ~4095
INT_MAX = 2147483647


def _tc_body(qT_ref, vT_ref, cpos_ref, codesT_ref, shT_ref, sel_ref,
             out_c_ref, out_s_ref):
    qT = qT_ref[...]
    cpos = cpos_ref[...]
    acc = None
    for d in range(3):
        diff = cpos[:, d:d + 1] - qT[d:d + 1, :]
        acc = diff * diff if acc is None else acc + diff * diff
    mm = jnp.dot(cpos.astype(jnp.bfloat16), qT.astype(jnp.bfloat16),
                 preferred_element_type=jnp.float32)
    qq = jnp.sum(qT * qT, axis=0, keepdims=True)
    cc = jnp.sum(cpos * cpos, axis=1, keepdims=True)
    d2sel = qq + cc - 2.0 * mm
    iota = jax.lax.broadcasted_iota(jnp.int32, (NUM_CODES, QBLK), 0)
    p = jnp.bitwise_or(
        jnp.bitwise_and(jax.lax.bitcast_convert_type(d2sel, jnp.int32),
                        IDX_MASK),
        iota)

    m = jnp.min(p, axis=0, keepdims=True)

    def step(_, m):
        return jnp.min(jnp.where(p > m, p, INT_MAX), axis=0, keepdims=True)

    m = jax.lax.fori_loop(0, NUM_NEIGHBORS - 1, step, m)

    WT = jnp.where(p <= m, 1.0 / (acc + 1e-16), 0.0)
    wsum = jnp.sum(WT, axis=0, keepdims=True)

    qcT = jnp.dot(codesT_ref[...], WT,
                  preferred_element_type=jnp.float32) / wsum
    GT = jnp.dot(shT_ref[...], WT,
                 preferred_element_type=jnp.float32) / wsum

    vT = vT_ref[...]
    x = vT[0:1, :]
    y = vT[1:2, :]
    z = vT[2:3, :]
    xx, yy, zz = x * x, y * y, z * z
    shb = [
        jnp.full((1, QBLK), SH_C0, jnp.float32),
        -SH_C1 * y,
        SH_C1 * z,
        -SH_C1 * x,
        SH_C2[0] * (x * y),
        SH_C2[1] * (y * z),
        SH_C2[2] * (2.0 * zz - xx - yy),
        SH_C2[3] * (x * z),
        SH_C2[4] * (xx - yy),
    ]
    bidx = jax.lax.broadcasted_iota(jnp.int32, (SH_WIDTH, QBLK), 0) % SH_BASIS_DIM
    MT = jnp.zeros((SH_WIDTH, QBLK), jnp.float32)
    for b in range(SH_BASIS_DIM):
        MT = jnp.where(bidx == b, shb[b], MT)
    out_s_ref[...] = jnp.dot(sel_ref[...], GT * MT,
                             preferred_element_type=jnp.float32)
    out_c_ref[...] = qcT


def kernel(codes_position, codes, sh_codes, indices, query_points, viewdirs):
    idx0 = indices[0]
    cpos = codes_position[idx0]
    codesT = codes[idx0].T
    shT = sh_codes[idx0].T
    qT = query_points[0].T
    vT = viewdirs.T

    sel_np = np.zeros((CODE_DIM, SH_WIDTH), np.float32)
    sel_np[np.arange(SH_WIDTH) // SH_BASIS_DIM, np.arange(SH_WIDTH)] = 1.0
    sel = jnp.asarray(sel_np)

    grid = (NUM_POINTS // QBLK,)
    out_cT, out_sT = pl.pallas_call(
        _tc_body,
        grid=grid,
        in_specs=[
            pl.BlockSpec((3, QBLK), lambda i: (0, i)),
            pl.BlockSpec((3, QBLK), lambda i: (0, i)),
            pl.BlockSpec((NUM_CODES, 3), lambda i: (0, 0)),
            pl.BlockSpec((CODE_DIM, NUM_CODES), lambda i: (0, 0)),
            pl.BlockSpec((SH_WIDTH, NUM_CODES), lambda i: (0, 0)),
            pl.BlockSpec((CODE_DIM, SH_WIDTH), lambda i: (0, 0)),
        ],
        out_specs=[
            pl.BlockSpec((CODE_DIM, QBLK), lambda i: (0, i)),
            pl.BlockSpec((CODE_DIM, QBLK), lambda i: (0, i)),
        ],
        out_shape=[
            jax.ShapeDtypeStruct((CODE_DIM, NUM_POINTS), jnp.float32),
            jax.ShapeDtypeStruct((CODE_DIM, NUM_POINTS), jnp.float32),
        ],
    )(qT, vT, cpos, codesT, shT, sel)
    return (out_cT.T, out_sT.T)


# speedup vs baseline: 2.1378x; 1.0145x over previous
import jax
import jax.numpy as jnp
import numpy as np
from jax.experimental import pallas as pl

SH_C0 = 0.28209479177387814
SH_C1 = 0.4886025119029199
SH_C2 = [1.0925484305920792, -1.0925484305920792, 0.31539156525252005,
         -1.0925484305920792, 0.5462742152960396]

NUM_CODES = 4096
CODE_DIM = 32
NUM_NEIGHBORS = 16
SH_BASIS_DIM = 9
SH_WIDTH = CODE_DIM * SH_BASIS_DIM
NUM_POINTS = 8192
QBLK = 256
IDX_MASK = ~4095
INT_MAX = 2147483647


def _tc_body(qT_ref, vT_ref, cpos_ref, codesT_ref, shT_ref, sel_ref,
             out_c_ref, out_s_ref):
    qT = qT_ref[...]
    cpos = cpos_ref[...]
    mm = jnp.dot(cpos.astype(jnp.bfloat16), qT.astype(jnp.bfloat16),
                 preferred_element_type=jnp.float32)
    qq = jnp.sum(qT * qT, axis=0, keepdims=True)
    cc = jnp.sum(cpos * cpos, axis=1, keepdims=True)
    d2sel = qq + cc - 2.0 * mm
    iota = jax.lax.broadcasted_iota(jnp.int32, (NUM_CODES, QBLK), 0)
    p = jnp.bitwise_or(
        jnp.bitwise_and(jax.lax.bitcast_convert_type(d2sel, jnp.int32),
                        IDX_MASK),
        iota)

    m = jnp.min(p, axis=0, keepdims=True)

    def step(_, m):
        return jnp.min(jnp.where(p > m, p, INT_MAX), axis=0, keepdims=True)

    m = jax.lax.fori_loop(0, NUM_NEIGHBORS - 1, step, m)

    acc = None
    for d in range(3):
        diff = cpos[:, d:d + 1] - qT[d:d + 1, :]
        acc = diff * diff if acc is None else acc + diff * diff
    WT = jnp.where(p <= m, 1.0 / (acc + 1e-16), 0.0)
    wsum = jnp.sum(WT, axis=0, keepdims=True)

    qcT = jnp.dot(codesT_ref[...], WT,
                  preferred_element_type=jnp.float32) / wsum
    GT = jnp.dot(shT_ref[...], WT,
                 preferred_element_type=jnp.float32) / wsum

    vT = vT_ref[...]
    x = vT[0:1, :]
    y = vT[1:2, :]
    z = vT[2:3, :]
    xx, yy, zz = x * x, y * y, z * z
    shb = [
        jnp.full((1, QBLK), SH_C0, jnp.float32),
        -SH_C1 * y,
        SH_C1 * z,
        -SH_C1 * x,
        SH_C2[0] * (x * y),
        SH_C2[1] * (y * z),
        SH_C2[2] * (2.0 * zz - xx - yy),
        SH_C2[3] * (x * z),
        SH_C2[4] * (xx - yy),
    ]
    bidx = jax.lax.broadcasted_iota(jnp.int32, (SH_WIDTH, QBLK), 0) % SH_BASIS_DIM
    MT = jnp.zeros((SH_WIDTH, QBLK), jnp.float32)
    for b in range(SH_BASIS_DIM):
        MT = jnp.where(bidx == b, shb[b], MT)
    out_s_ref[...] = jnp.dot(sel_ref[...], GT * MT,
                             preferred_element_type=jnp.float32)
    out_c_ref[...] = qcT


def kernel(codes_position, codes, sh_codes, indices, query_points, viewdirs):
    idx0 = indices[0]
    cpos = codes_position[idx0]
    codesT = codes[idx0].T
    shT = sh_codes[idx0].T
    qT = query_points[0].T
    vT = viewdirs.T

    sel_np = np.zeros((CODE_DIM, SH_WIDTH), np.float32)
    sel_np[np.arange(SH_WIDTH) // SH_BASIS_DIM, np.arange(SH_WIDTH)] = 1.0
    sel = jnp.asarray(sel_np)

    grid = (NUM_POINTS // QBLK,)
    out_cT, out_sT = pl.pallas_call(
        _tc_body,
        grid=grid,
        in_specs=[
            pl.BlockSpec((3, QBLK), lambda i: (0, i)),
            pl.BlockSpec((3, QBLK), lambda i: (0, i)),
            pl.BlockSpec((NUM_CODES, 3), lambda i: (0, 0)),
            pl.BlockSpec((CODE_DIM, NUM_CODES), lambda i: (0, 0)),
            pl.BlockSpec((SH_WIDTH, NUM_CODES), lambda i: (0, 0)),
            pl.BlockSpec((CODE_DIM, SH_WIDTH), lambda i: (0, 0)),
        ],
        out_specs=[
            pl.BlockSpec((CODE_DIM, QBLK), lambda i: (0, i)),
            pl.BlockSpec((CODE_DIM, QBLK), lambda i: (0, i)),
        ],
        out_shape=[
            jax.ShapeDtypeStruct((CODE_DIM, NUM_POINTS), jnp.float32),
            jax.ShapeDtypeStruct((CODE_DIM, NUM_POINTS), jnp.float32),
        ],
    )(qT, vT, cpos, codesT, shT, sel)
    return (out_cT.T, out_sT.T)
